# Initial kernel scaffold; baseline (speedup 1.0000x reference)
import jax
import jax.numpy as jnp
from jax.experimental import pallas as pl


def kernel(inputs, som_weights, locations):
    raise NotImplementedError("write your pallas kernel here")


# speedup vs baseline: 18.5373x; 18.5373x over previous
import jax
import jax.numpy as jnp
from jax import lax
from jax.experimental import pallas as pl

M, N, DIM = 32, 32, 256
K = M * N
B = 256
T2_INV = 1.0 / (100.0 * 100.0)


def _som_body(x_ref, wt_ref, li_ref, lj_ref, loss_ref):
    x = x_ref[...]
    wt = wt_ref[...]
    li = li_ref[...]
    lj = lj_ref[...]

    xw = lax.dot_general(
        x, wt, (((1,), (0,)), ((), ())),
        preferred_element_type=jnp.float32,
        precision=lax.Precision.HIGHEST,
    )
    w2 = jnp.sum(wt * wt, axis=0, keepdims=True)
    x2 = jnp.sum(x * x, axis=1, keepdims=True)
    score = w2 - 2.0 * xw
    dist = score + x2

    minval = jnp.min(score, axis=1, keepdims=True)
    kio = lax.broadcasted_iota(jnp.int32, (B, K), 1)
    bmu = jnp.min(jnp.where(score == minval, kio, K), axis=1, keepdims=True)

    onehot = (kio == bmu).astype(jnp.float32)
    bi = jnp.sum(onehot * li, axis=1, keepdims=True)
    bj = jnp.sum(onehot * lj, axis=1, keepdims=True)

    man = jnp.abs(li - bi) + jnp.abs(lj - bj)
    infl = jnp.exp(-(man * man) * T2_INV)
    rowsum = jnp.sum(dist * infl, axis=1, keepdims=True)
    loss_ref[...] = jnp.sum(rowsum, axis=0, keepdims=True) * (1.0 / N)


def kernel(inputs, som_weights, locations):
    wt = som_weights.T
    li = locations[:, 0].reshape(1, K)
    lj = locations[:, 1].reshape(1, K)
    loss = pl.pallas_call(
        _som_body,
        out_shape=jax.ShapeDtypeStruct((1, 1), jnp.float32),
    )(inputs, wt, li, lj)
    return som_weights, loss.reshape(())
